# baseline (speedup 1.0000x reference)
import functools

import jax
import jax.numpy as jnp
import numpy as np
from jax.experimental import pallas as pl

NU = 25001
NI = 25001
D = 64
LAYERS = 2
NB = 3
REG = 0.001


def _lightgcn(x, edge_index, n_users, layers):
    n = x.shape[0]
    src_u = edge_index[0]
    dst_i = edge_index[1] + n_users
    src = jnp.concatenate([src_u, dst_i])
    dst = jnp.concatenate([dst_i, src_u])
    deg = jnp.zeros((n,), dtype=x.dtype).at[src].add(1.0)
    deg = jnp.where(deg > 0, deg, 1.0)
    norm = (1.0 / jnp.sqrt(deg[src])) * (1.0 / jnp.sqrt(deg[dst]))
    embs = [x]
    h = x
    for _ in range(layers):
        msg = h[src] * norm[:, None]
        h = jnp.zeros_like(h).at[dst].add(msg)
        embs.append(h)
    return jnp.mean(jnp.stack(embs, axis=0), axis=0)


def _mutual_attention(fe, d):
    Bb = fe.shape[1]
    table = []
    feT = jnp.swapaxes(fe, -1, -2)
    for i in range(Bb):
        be = fe[:, i:i + 1, :]
        table.append(jnp.matmul(be, feT))
    last = table[-1]
    norm_num = jnp.sum(last ** 2, axis=1) + 1e-12
    scores = []
    for i in range(Bb - 1):
        res = jnp.sum(last * table[i], axis=1, keepdims=True) * last
        clear = res / norm_num[:, None, :]
        scores.append(clear)
    scores_all = jnp.concatenate(scores, axis=-2)
    s = jnp.sum(jnp.concatenate(scores, axis=-2), axis=-2)[:, None, :] + last
    scores_all = jnp.concatenate([scores_all, s], axis=1)
    att = jax.nn.softmax(scores_all / np.sqrt(d), axis=-1)
    return jnp.matmul(att, fe)


def _identity_kernel(x_ref, o_ref):
    o_ref[...] = x_ref[...]


def kernel(user_emb, item_emb, W, item_behaviour_degree, batch_data,
           edge_index_global, edge_index_b0, edge_index_b1, edge_index_b2):
    all_emb = jnp.concatenate([user_emb, item_emb], axis=0)
    all_emb = pl.pallas_call(
        _identity_kernel,
        out_shape=jax.ShapeDtypeStruct(all_emb.shape, all_emb.dtype),
    )(all_emb)
    all_emb = _lightgcn(all_emb, edge_index_global, NU, LAYERS)
    ue_list, ie_list = [], []
    for ei in (edge_index_b0, edge_index_b1, edge_index_b2):
        be = _lightgcn(all_emb, ei, NU, LAYERS)
        ue_list.append(be[:NU])
        ie_list.append(be[NU:])
    all_user = jnp.stack(ue_list, axis=1)
    all_item = jnp.stack(ie_list, axis=1)
    all_user = _mutual_attention(all_user, D)
    weight = item_behaviour_degree * W
    weight = weight / (jnp.sum(weight, axis=1, keepdims=True) + 1e-08)
    all_item = jnp.sum(all_item * weight[:, :, None], axis=1)
    total_loss1 = 0.0
    for i in range(NB):
        data = batch_data[:, i]
        users = data[:, 0]
        items = data[:, 1:]
        user_feature = all_user[:, i][users][:, None, :]
        item_feature = all_item[items]
        scores = jnp.sum(user_feature * item_feature, axis=2)
        pos, neg = scores[:, 0], scores[:, 1]
        total_loss1 = total_loss1 + (-jnp.mean(jax.nn.log_sigmoid(pos - neg)))
    total_loss = total_loss1 + REG * (
        (jnp.linalg.norm(user_emb) + jnp.linalg.norm(item_emb)) / item_emb.shape[0])
    return total_loss


# speedup vs baseline: 14.8899x; 14.8899x over previous
import jax
import jax.numpy as jnp
import numpy as np
from jax import lax
from jax.experimental import pallas as pl
from jax.experimental.pallas import tpu as pltpu
from jax.experimental.pallas import tpu_sc as plsc

NU = 25001
NI = 25001
N = NU + NI
D = 64
Q = 16
NQ = 4
NB = 3
REG = 0.001

N_PAD = 51200
RPS = N_PAD // 16
WB = 800
C = 1024
NSUB = C // 128

EPAD_GLOBAL = 2424832
EPAD_BEHAV = 819200


def _deg_body(e0, e1, e2, e3, ones16, zeros16, out_hbm,
              idxb, onesb, zerob, tbuf, acc, sem):
    c = lax.axis_index("c")
    s = lax.axis_index("s")
    w = c * 16 + s
    pltpu.sync_copy(ones16, onesb)
    pltpu.sync_copy(zeros16, zerob)
    for g, eref in enumerate((e0, e1, e2, e3)):
        for j in range(RPS // WB):
            pltpu.sync_copy(zerob, acc.at[pl.ds((s * (RPS // WB) + j) * WB, WB)])
        plsc.subcore_barrier()
        rows_pw = eref.shape[0] // 32
        nch = rows_pw // NSUB

        def chunk(i, _, eref=eref, base=w * rows_pw):
            pltpu.sync_copy(eref.at[pl.ds(base + i * NSUB, NSUB)], idxb)
            hs = [
                pltpu.async_copy(onesb, acc.at[idxb.at[j]], sem, add=True)
                for j in range(NSUB)
            ]
            for h in hs:
                h.wait()
            return 0

        lax.fori_loop(0, nch, chunk, 0)
        plsc.subcore_barrier()
        ob = (g * 2 + c) * N_PAD
        for j in range(RPS // WB):
            r0 = (s * (RPS // WB) + j) * WB
            pltpu.sync_copy(acc.at[pl.ds(r0, WB)], tbuf)
            pltpu.sync_copy(tbuf, out_hbm.at[pl.ds(ob + r0, WB)])
        plsc.subcore_barrier()


def _make_deg_kernel():
    return pl.kernel(
        _deg_body,
        out_type=jax.ShapeDtypeStruct((8 * N_PAD, Q), jnp.float32),
        mesh=plsc.VectorSubcoreMesh(core_axis_name="c", subcore_axis_name="s"),
        scratch_types=[
            pltpu.VMEM((NSUB, 128), jnp.int32),
            pltpu.VMEM((128, Q), jnp.float32),
            pltpu.VMEM((WB, Q), jnp.float32),
            pltpu.VMEM((WB, Q), jnp.float32),
            pltpu.VMEM_SHARED((N_PAD, Q), jnp.float32),
            pltpu.SemaphoreType.DMA,
        ],
        compiler_params=pltpu.CompilerParams(use_tc_tiling_on_sc=False),
        name="gcn_degrees",
    )


def _prop_body(x_hbm, src2, dst2, zeros16, out_hbm,
               sidx, didx, gbuf, zbuf, acc, gsem, ssem):
    c = lax.axis_index("c")
    s = lax.axis_index("s")
    pltpu.sync_copy(zeros16, zbuf)
    erows = dst2.shape[0]
    rows_ps = erows // 16
    nch = rows_ps // NSUB
    base = s * rows_ps
    for p in range(2):
        qq = c * 2 + p
        off = qq * N_PAD
        for j in range(RPS // WB):
            pltpu.sync_copy(zbuf, acc.at[pl.ds((s * (RPS // WB) + j) * WB, WB)])
        plsc.subcore_barrier()

        def chunk(i, _, off=off):
            er = base + i * NSUB
            pltpu.sync_copy(src2.at[pl.ds(er, NSUB)], sidx)
            pltpu.sync_copy(dst2.at[pl.ds(er, NSUB)], didx)
            for r in range(NSUB):
                for k in range(8):
                    v = sidx[r, pl.ds(k * 16, 16)]
                    sidx[r, pl.ds(k * 16, 16)] = v + off
            gs = [
                pltpu.async_copy(x_hbm.at[sidx.at[j]],
                                 gbuf.at[pl.ds(j * 128, 128)], gsem)
                for j in range(NSUB)
            ]
            for h in gs:
                h.wait()
            ss = [
                pltpu.async_copy(gbuf.at[pl.ds(j * 128, 128)],
                                 acc.at[didx.at[j]], ssem, add=True)
                for j in range(NSUB)
            ]
            for h in ss:
                h.wait()
            return 0

        lax.fori_loop(0, nch, chunk, 0)
        plsc.subcore_barrier()
        for j in range(RPS // WB):
            r0 = (s * (RPS // WB) + j) * WB
            pltpu.sync_copy(acc.at[pl.ds(r0, WB)], zbuf)
            pltpu.sync_copy(zbuf, out_hbm.at[pl.ds(off + r0, WB)])
            pltpu.sync_copy(zeros16, zbuf)
        plsc.subcore_barrier()


def _make_prop_kernel():
    return pl.kernel(
        _prop_body,
        out_type=jax.ShapeDtypeStruct((NQ * N_PAD, Q), jnp.float32),
        mesh=plsc.VectorSubcoreMesh(core_axis_name="c", subcore_axis_name="s"),
        scratch_types=[
            pltpu.VMEM((NSUB, 128), jnp.int32),
            pltpu.VMEM((NSUB, 128), jnp.int32),
            pltpu.VMEM((C, Q), jnp.float32),
            pltpu.VMEM((WB, Q), jnp.float32),
            pltpu.VMEM_SHARED((N_PAD, Q), jnp.float32),
            pltpu.SemaphoreType.DMA,
            pltpu.SemaphoreType.DMA,
        ],
        compiler_params=pltpu.CompilerParams(use_tc_tiling_on_sc=False),
        name="gcn_propagate",
    )


def _build_edges(ei, epad):
    s = ei[0].astype(jnp.int32)
    d = ei[1].astype(jnp.int32) + NU
    src = jnp.concatenate([s, d])
    dst = jnp.concatenate([d, s])
    pad = epad - src.shape[0]
    i = jnp.arange(pad, dtype=jnp.int32)
    psrc = (i * 97) % N
    pdst = N + (i % (N_PAD - N))
    src = jnp.concatenate([src, psrc]).reshape(epad // 128, 128)
    dst = jnp.concatenate([dst, pdst]).reshape(epad // 128, 128)
    return src, dst


def _to_quarter(x_pad):
    return x_pad.reshape(N_PAD, NQ, Q).transpose(1, 0, 2).reshape(NQ * N_PAD, Q)


def _from_quarter(xq):
    return xq.reshape(NQ, N_PAD, Q).transpose(1, 0, 2).reshape(N_PAD, D)


def _lightgcn_sc(prop, xq, src2, dst2, dsq, zeros16):
    u0 = xq * dsq
    a1 = prop(u0, src2, dst2, zeros16)
    u1 = a1 * (dsq * dsq)
    a2 = prop(u1, src2, dst2, zeros16)
    return (xq + a1 * dsq + a2 * dsq) / 3.0


def _mutual_attention(fe, d):
    Bb = fe.shape[1]
    table = []
    feT = jnp.swapaxes(fe, -1, -2)
    for i in range(Bb):
        be = fe[:, i:i + 1, :]
        table.append(jnp.matmul(be, feT))
    last = table[-1]
    norm_num = jnp.sum(last ** 2, axis=1) + 1e-12
    scores = []
    for i in range(Bb - 1):
        res = jnp.sum(last * table[i], axis=1, keepdims=True) * last
        clear = res / norm_num[:, None, :]
        scores.append(clear)
    scores_all = jnp.concatenate(scores, axis=-2)
    s = jnp.sum(jnp.concatenate(scores, axis=-2), axis=-2)[:, None, :] + last
    scores_all = jnp.concatenate([scores_all, s], axis=1)
    att = jax.nn.softmax(scores_all / np.sqrt(d), axis=-1)
    return jnp.matmul(att, fe)


def kernel(user_emb, item_emb, W, item_behaviour_degree, batch_data,
           edge_index_global, edge_index_b0, edge_index_b1, edge_index_b2):
    deg_kernel = _make_deg_kernel()
    prop = _make_prop_kernel()

    x = jnp.concatenate([user_emb, item_emb], axis=0)
    x_pad = jnp.zeros((N_PAD, D), jnp.float32).at[:N].set(x)
    xq = _to_quarter(x_pad)

    edges = [
        _build_edges(edge_index_global, EPAD_GLOBAL),
        _build_edges(edge_index_b0, EPAD_BEHAV),
        _build_edges(edge_index_b1, EPAD_BEHAV),
        _build_edges(edge_index_b2, EPAD_BEHAV),
    ]

    ones16 = jnp.ones((128, Q), jnp.float32)
    zeros16 = jnp.zeros((WB, Q), jnp.float32)

    degs = deg_kernel(edges[0][1], edges[1][1], edges[2][1], edges[3][1],
                      ones16, zeros16)
    degs = degs.reshape(4, 2, N_PAD, Q)
    dsqs = []
    for g in range(4):
        deg = degs[g, 0, :, 0] + degs[g, 1, :, 0]
        deg = jnp.where(deg > 0, deg, 1.0)
        ds = lax.rsqrt(deg)
        dsqs.append(jnp.tile(ds, NQ)[:, None])

    gq = _lightgcn_sc(prop, xq, edges[0][0], edges[0][1], dsqs[0], zeros16)
    behs = [
        _lightgcn_sc(prop, gq, edges[g + 1][0], edges[g + 1][1],
                     dsqs[g + 1], zeros16)
        for g in range(NB)
    ]

    be_full = [_from_quarter(bq)[:N] for bq in behs]
    all_user = jnp.stack([bf[:NU] for bf in be_full], axis=1)
    all_item = jnp.stack([bf[NU:] for bf in be_full], axis=1)

    all_user = _mutual_attention(all_user, D)
    weight = item_behaviour_degree * W
    weight = weight / (jnp.sum(weight, axis=1, keepdims=True) + 1e-08)
    all_item = jnp.sum(all_item * weight[:, :, None], axis=1)

    total_loss1 = 0.0
    for i in range(NB):
        data = batch_data[:, i]
        users = data[:, 0]
        items = data[:, 1:]
        user_feature = all_user[:, i][users][:, None, :]
        item_feature = all_item[items]
        scores = jnp.sum(user_feature * item_feature, axis=2)
        pos, neg = scores[:, 0], scores[:, 1]
        total_loss1 = total_loss1 + (-jnp.mean(jax.nn.log_sigmoid(pos - neg)))
    total_loss = total_loss1 + REG * (
        (jnp.linalg.norm(user_emb) + jnp.linalg.norm(item_emb))
        / item_emb.shape[0])
    return total_loss


# speedup vs baseline: 21.1748x; 1.4221x over previous
import jax
import jax.numpy as jnp
import numpy as np
from jax import lax
from jax.experimental import pallas as pl
from jax.experimental.pallas import tpu as pltpu
from jax.experimental.pallas import tpu_sc as plsc

NU = 25001
NI = 25001
N = NU + NI
D = 64
Q = 16
NQ = 4
NB = 3
REG = 0.001

N_PAD = 51200
RPS = N_PAD // 16
WB = 800
C = 1024
NSUB = C // 128

EPAD_GLOBAL = 2424832
EPAD_BEHAV = 819200


def _deg_body(e0, e1, e2, e3, ones128, zeros16, out_hbm,
              idxa, idxb, onesb3, zerob, tbuf, acc, sema, semb):
    c = lax.axis_index("c")
    s = lax.axis_index("s")
    w = c * 16 + s
    for r in range(NSUB):
        pltpu.sync_copy(ones128, onesb3.at[pl.ds(r * 128, 128)])
    pltpu.sync_copy(zeros16, zerob)
    for g, eref in enumerate((e0, e1, e2, e3)):
        for j in range(RPS // WB):
            pltpu.sync_copy(zerob, acc.at[pl.ds((s * (RPS // WB) + j) * WB, WB)])
        plsc.subcore_barrier()
        epw = eref.shape[0] // 32
        nch = epw // C
        base = w * epw

        def load(i, buf, eref=eref, base=base):
            pltpu.sync_copy(eref.at[pl.ds(base + i * C, C)], buf)

        load(0, idxa)
        M = nch // 2

        def pair(i, _):
            ha = pltpu.async_copy(onesb3, acc.at[idxa], sema, add=True)
            load(2 * i + 1, idxb)
            ha.wait()
            hb = pltpu.async_copy(onesb3, acc.at[idxb], semb, add=True)
            load(jnp.minimum(2 * i + 2, nch - 1), idxa)
            hb.wait()
            return 0

        lax.fori_loop(0, M, pair, 0)
        if nch % 2 == 1:
            pltpu.async_copy(onesb3, acc.at[idxa], sema, add=True).wait()
        plsc.subcore_barrier()
        ob = (g * 2 + c) * N_PAD
        for j in range(RPS // WB):
            r0 = (s * (RPS // WB) + j) * WB
            pltpu.sync_copy(acc.at[pl.ds(r0, WB)], tbuf)
            pltpu.sync_copy(tbuf, out_hbm.at[pl.ds(ob + r0, WB)])
        plsc.subcore_barrier()


def _make_deg_kernel():
    return pl.kernel(
        _deg_body,
        out_type=jax.ShapeDtypeStruct((8 * N_PAD, Q), jnp.float32),
        mesh=plsc.VectorSubcoreMesh(core_axis_name="c", subcore_axis_name="s"),
        scratch_types=[
            pltpu.VMEM((C,), jnp.int32),
            pltpu.VMEM((C,), jnp.int32),
            pltpu.VMEM((C, Q), jnp.float32),
            pltpu.VMEM((WB, Q), jnp.float32),
            pltpu.VMEM((WB, Q), jnp.float32),
            pltpu.VMEM_SHARED((N_PAD, Q), jnp.float32),
            pltpu.SemaphoreType.DMA,
            pltpu.SemaphoreType.DMA,
        ],
        compiler_params=pltpu.CompilerParams(use_tc_tiling_on_sc=False),
        name="gcn_degrees",
    )


def _prop_body(x_hbm, src2, dst2, zeros16, out_hbm,
               sidxa, didxa, sidxb, didxb, gbufa, gbufb, zbuf, tbuf, acc,
               gsema, gsemb, ssema, ssemb):
    c = lax.axis_index("c")
    s = lax.axis_index("s")
    pltpu.sync_copy(zeros16, zbuf)
    eps = dst2.shape[0] // 16
    nch = eps // C
    base = s * eps
    for p in range(2):
        qq = c * 2 + p
        off = qq * N_PAD
        for j in range(RPS // WB):
            pltpu.sync_copy(zbuf, acc.at[pl.ds((s * (RPS // WB) + j) * WB, WB)])
        plsc.subcore_barrier()

        def load_idx(i, sidx, didx, off=off):
            eb = base + i * C
            pltpu.sync_copy(src2.at[pl.ds(eb, C)], sidx)
            pltpu.sync_copy(dst2.at[pl.ds(eb, C)], didx)
            for k in range(C // 16):
                v = sidx[pl.ds(k * 16, 16)]
                sidx[pl.ds(k * 16, 16)] = v + off

        load_idx(0, sidxa, didxa)
        ga0 = pltpu.async_copy(x_hbm.at[sidxa], gbufa, gsema)
        M = nch // 2

        def pair(i, _):
            @pl.when(i > 0)
            def _():
                pltpu.make_async_copy(gbufb, acc.at[didxb], ssemb).wait()
            load_idx(2 * i + 1, sidxb, didxb)
            gb = pltpu.async_copy(x_hbm.at[sidxb], gbufb, gsemb)
            pltpu.make_async_copy(x_hbm.at[sidxa], gbufa, gsema).wait()
            sa = pltpu.async_copy(gbufa, acc.at[didxa], ssema, add=True)
            sa.wait()
            load_idx(jnp.minimum(2 * i + 2, nch - 2), sidxa, didxa)
            pltpu.async_copy(x_hbm.at[sidxa], gbufa, gsema)
            gb.wait()
            pltpu.async_copy(gbufb, acc.at[didxb], ssemb, add=True)
            return 0

        lax.fori_loop(0, M, pair, 0)
        pltpu.make_async_copy(x_hbm.at[sidxa], gbufa, gsema).wait()
        pltpu.make_async_copy(gbufb, acc.at[didxb], ssemb).wait()
        plsc.subcore_barrier()
        for j in range(RPS // WB):
            r0 = (s * (RPS // WB) + j) * WB
            pltpu.sync_copy(acc.at[pl.ds(r0, WB)], tbuf)
            pltpu.sync_copy(tbuf, out_hbm.at[pl.ds(off + r0, WB)])
        plsc.subcore_barrier()


def _make_prop_kernel():
    return pl.kernel(
        _prop_body,
        out_type=jax.ShapeDtypeStruct((NQ * N_PAD, Q), jnp.float32),
        mesh=plsc.VectorSubcoreMesh(core_axis_name="c", subcore_axis_name="s"),
        scratch_types=[
            pltpu.VMEM((C,), jnp.int32),
            pltpu.VMEM((C,), jnp.int32),
            pltpu.VMEM((C,), jnp.int32),
            pltpu.VMEM((C,), jnp.int32),
            pltpu.VMEM((C, Q), jnp.float32),
            pltpu.VMEM((C, Q), jnp.float32),
            pltpu.VMEM((WB, Q), jnp.float32),
            pltpu.VMEM((WB, Q), jnp.float32),
            pltpu.VMEM_SHARED((N_PAD, Q), jnp.float32),
            pltpu.SemaphoreType.DMA,
            pltpu.SemaphoreType.DMA,
            pltpu.SemaphoreType.DMA,
            pltpu.SemaphoreType.DMA,
        ],
        compiler_params=pltpu.CompilerParams(use_tc_tiling_on_sc=False),
        name="gcn_propagate",
    )


def _build_edges(ei, epad):
    s = ei[0].astype(jnp.int32)
    d = ei[1].astype(jnp.int32) + NU
    src = jnp.concatenate([s, d])
    dst = jnp.concatenate([d, s])
    pad = epad - src.shape[0]
    i = jnp.arange(pad, dtype=jnp.int32)
    psrc = (i * 97) % N
    pdst = N + (i % (N_PAD - N))
    src = jnp.concatenate([src, psrc])
    dst = jnp.concatenate([dst, pdst])
    return src, dst


def _to_quarter(x_pad):
    return x_pad.reshape(N_PAD, NQ, Q).transpose(1, 0, 2).reshape(NQ * N_PAD, Q)


def _from_quarter(xq):
    return xq.reshape(NQ, N_PAD, Q).transpose(1, 0, 2).reshape(N_PAD, D)


def _lightgcn_sc(prop, xq, src2, dst2, dsq, zeros16):
    u0 = xq * dsq
    a1 = prop(u0, src2, dst2, zeros16)
    u1 = a1 * (dsq * dsq)
    a2 = prop(u1, src2, dst2, zeros16)
    return (xq + a1 * dsq + a2 * dsq) / 3.0


def _mutual_attention(fe, d):
    Bb = fe.shape[1]
    table = []
    feT = jnp.swapaxes(fe, -1, -2)
    for i in range(Bb):
        be = fe[:, i:i + 1, :]
        table.append(jnp.matmul(be, feT))
    last = table[-1]
    norm_num = jnp.sum(last ** 2, axis=1) + 1e-12
    scores = []
    for i in range(Bb - 1):
        res = jnp.sum(last * table[i], axis=1, keepdims=True) * last
        clear = res / norm_num[:, None, :]
        scores.append(clear)
    scores_all = jnp.concatenate(scores, axis=-2)
    s = jnp.sum(jnp.concatenate(scores, axis=-2), axis=-2)[:, None, :] + last
    scores_all = jnp.concatenate([scores_all, s], axis=1)
    att = jax.nn.softmax(scores_all / np.sqrt(d), axis=-1)
    return jnp.matmul(att, fe)


def kernel(user_emb, item_emb, W, item_behaviour_degree, batch_data,
           edge_index_global, edge_index_b0, edge_index_b1, edge_index_b2):
    deg_kernel = _make_deg_kernel()
    prop = _make_prop_kernel()

    x = jnp.concatenate([user_emb, item_emb], axis=0)
    x_pad = jnp.zeros((N_PAD, D), jnp.float32).at[:N].set(x)
    xq = _to_quarter(x_pad)

    edges = [
        _build_edges(edge_index_global, EPAD_GLOBAL),
        _build_edges(edge_index_b0, EPAD_BEHAV),
        _build_edges(edge_index_b1, EPAD_BEHAV),
        _build_edges(edge_index_b2, EPAD_BEHAV),
    ]

    ones16 = jnp.ones((128, Q), jnp.float32)
    zeros16 = jnp.zeros((WB, Q), jnp.float32)

    degs = deg_kernel(edges[0][1], edges[1][1], edges[2][1], edges[3][1],
                      ones16, zeros16)
    degs = degs.reshape(4, 2, N_PAD, Q)
    dsqs = []
    for g in range(4):
        deg = degs[g, 0, :, 0] + degs[g, 1, :, 0]
        deg = jnp.where(deg > 0, deg, 1.0)
        ds = lax.rsqrt(deg)
        dsqs.append(jnp.tile(ds, NQ)[:, None])

    gq = _lightgcn_sc(prop, xq, edges[0][0], edges[0][1], dsqs[0], zeros16)
    behs = [
        _lightgcn_sc(prop, gq, edges[g + 1][0], edges[g + 1][1],
                     dsqs[g + 1], zeros16)
        for g in range(NB)
    ]

    be_full = [_from_quarter(bq)[:N] for bq in behs]
    all_user = jnp.stack([bf[:NU] for bf in be_full], axis=1)
    all_item = jnp.stack([bf[NU:] for bf in be_full], axis=1)

    all_user = _mutual_attention(all_user, D)
    weight = item_behaviour_degree * W
    weight = weight / (jnp.sum(weight, axis=1, keepdims=True) + 1e-08)
    all_item = jnp.sum(all_item * weight[:, :, None], axis=1)

    total_loss1 = 0.0
    for i in range(NB):
        data = batch_data[:, i]
        users = data[:, 0]
        items = data[:, 1:]
        user_feature = all_user[:, i][users][:, None, :]
        item_feature = all_item[items]
        scores = jnp.sum(user_feature * item_feature, axis=2)
        pos, neg = scores[:, 0], scores[:, 1]
        total_loss1 = total_loss1 + (-jnp.mean(jax.nn.log_sigmoid(pos - neg)))
    total_loss = total_loss1 + REG * (
        (jnp.linalg.norm(user_emb) + jnp.linalg.norm(item_emb))
        / item_emb.shape[0])
    return total_loss
